# baseline (speedup 1.0000x reference)
import functools

import jax
import jax.numpy as jnp
from jax import lax
from jax.experimental import pallas as pl
from jax.experimental.pallas import tpu as pltpu
from jax.experimental.pallas import tpu_sc as plsc

_W = 512
_WSH = 9
_CAP = 4096
_RCAP = 240
_L = 16
_NB = 80
_TRASH = _NB - 1


def kernel(table_0, table_1, table_2, indices_0, indices_1, indices_2):
    B = indices_0.shape[0]
    V = table_0.shape[0]
    info = plsc.get_sparse_core_info()
    NC, NS = info.num_cores, info.num_subcores
    NW = NC * NS
    dims = (table_0.shape[1], table_1.shape[1], table_2.shape[1])
    padv = ((V + 127) // 128) * 128
    nwin = (V + _W - 1) // _W
    mesh = plsc.VectorSubcoreMesh(core_axis_name="c", subcore_axis_name="s")

    tT3 = [t.T.reshape(d // 8, 8, V)
           for t, d in zip((table_0, table_1, table_2), dims)]

    scratch = [
        pltpu.VMEM((8, 8, _W), jnp.float32),
        pltpu.VMEM((8, 8, _W), jnp.float32),
        pltpu.VMEM((_CAP,), jnp.int32),
        pltpu.VMEM((_CAP,), jnp.int32),
        pltpu.VMEM((_CAP,), jnp.int32),
        pltpu.VMEM((_NB + _L,), jnp.int32),
        pltpu.VMEM((_NB + _L,), jnp.int32),
        pltpu.VMEM((_NB + _L,), jnp.int32),
        pltpu.VMEM((_L,), jnp.int32),
        pltpu.VMEM((_RCAP + _L, 128), jnp.float32),
        pltpu.VMEM((_RCAP + _L,), jnp.int32),
        pltpu.VMEM((2048,), jnp.int32),
        pltpu.SemaphoreType.DMA,
        pltpu.SemaphoreType.DMA,
    ]

    @functools.partial(
        pl.kernel,
        mesh=mesh,
        out_type=tuple(
            jax.ShapeDtypeStruct((B, 128), jnp.float32) for _ in dims
        ),
        scratch_types=scratch,
        compiler_params=pltpu.CompilerParams(needs_layout_passes=False),
    )
    def gather3(t0, t1, t2, i0, i1, i2, o0, o1, o2,
                win0, win1, lanes, dests, order, hist, offs, curs, tmp,
                rows, rdst, ivb, s0, s1):
        wid = lax.axis_index("s") * NC + lax.axis_index("c")
        iota = lax.iota(jnp.int32, _L)
        wsem = (s0, s1)

        klo = nwin * wid // NW
        khi = nwin * (wid + 1) // NW

        def wstart(k):
            return jnp.minimum(k * _W, padv - _W)

        def runs(w_s):
            tmp[...] = w_s
            prev = plsc.load_gather(tmp, [jnp.maximum(iota - 1, 0)])
            nxt = plsc.load_gather(tmp, [jnp.minimum(iota + 1, _L - 1)])
            first = (iota == 0) | (w_s != prev)
            last = (iota == _L - 1) | (w_s != nxt)
            start = plsc.cummax(jnp.where(first, iota, 0))
            rank = iota - start
            return last, rank

        for n, (t, i, o, d) in enumerate(
            zip((t0, t1, t2), (i0, i1, i2), (o0, o1, o2), dims)
        ):
            nb = d // 8
            llo = wstart(klo)
            lhi = jnp.where(jnp.int32(khi) >= nwin, jnp.int32(V), khi * _W)

            def chunk_a(c8, n_hits, i=i):
                pltpu.sync_copy(i.at[pl.ds(c8 * 2048, 2048)], ivb)

                def grp_a(g, n_hits):
                    ivec = ivb[pl.ds(g * _L, _L)]
                    m = (ivec >= llo) & (ivec < lhi)
                    mi = m.astype(jnp.int32)
                    cs = plsc.cumsum(mi)
                    pos = n_hits + cs - mi
                    m = m & (pos < _CAP)
                    plsc.store_scatter(lanes, [pos], ivec, mask=m)
                    plsc.store_scatter(
                        dests, [pos], iota + (c8 * 2048 + g * _L), mask=m)
                    return n_hits + cs[_L - 1]

                return lax.fori_loop(0, 2048 // _L, grp_a, n_hits)

            n_hits = jnp.minimum(
                lax.fori_loop(0, B // 2048, chunk_a, jnp.int32(0)),
                jnp.int32(_CAP))
            ng = (n_hits + _L - 1) // _L

            def binit(q, _):
                hist[pl.ds(q * _L, _L)] = jnp.zeros((_L,), jnp.int32)
                return _
            lax.fori_loop(0, (_NB + _L) // _L, binit, 0)

            def whist(g, _):
                lvec = lanes[pl.ds(g * _L, _L)]
                valid = iota + g * _L < n_hits
                w = jnp.where(valid, (lvec - llo) >> _WSH, _TRASH)
                w_s, _src = plsc.sort_key_val(w, iota)
                last, rank = runs(w_s)
                plsc.addupdate_scatter(hist, [w_s], rank + 1, mask=last)
                return _
            lax.fori_loop(0, ng, whist, 0)

            def bscan(q, c):
                h = hist[pl.ds(q * _L, _L)]
                cs = plsc.cumsum(h)
                excl = cs - h + c
                offs[pl.ds(q * _L, _L)] = excl
                curs[pl.ds(q * _L, _L)] = excl
                return c + cs[_L - 1]
            lax.fori_loop(0, (_NB + _L) // _L, bscan, jnp.int32(0))

            def wplace(g, _):
                lvec = lanes[pl.ds(g * _L, _L)]
                valid = iota + g * _L < n_hits
                w = jnp.where(valid, (lvec - llo) >> _WSH, _TRASH)
                w_s, src = plsc.sort_key_val(w, iota + g * _L)
                last, rank = runs(w_s)
                base = plsc.load_gather(curs, [w_s])
                plsc.store_scatter(order, [base + rank], src,
                                   mask=w_s != _TRASH)
                plsc.addupdate_scatter(curs, [w_s], rank + 1, mask=last)
                return _
            lax.fori_loop(0, ng, wplace, 0)

            def rinit(q, _):
                rdst[pl.ds(q * _L, _L)] = jnp.full((_L,), -1, jnp.int32)
                return _
            lax.fori_loop(0, (_RCAP + _L) // _L, rinit, 0)

            pltpu.async_copy(
                t.at[:, :, pl.ds(wstart(klo), _W)],
                win0.at[pl.ds(0, nb)], wsem[0])

            def window_body(p, k, nl, t=t, o=o, d=d, nb=nb):
                wb = (win0, win1)[p]
                wn = (win0, win1)[p ^ 1]

                @pl.when(k + 1 < khi)
                def _():
                    pltpu.async_copy(
                        t.at[:, :, pl.ds(wstart(k + 1), _W)],
                        wn.at[pl.ds(0, nb)], wsem[p ^ 1])

                pltpu.make_async_copy(
                    t.at[:, :, pl.ds(0, _W)],
                    wb.at[pl.ds(0, nb)], wsem[p]).wait()
                ws = wstart(k)
                wl = jnp.full((_L,), k - klo, jnp.int32)
                start = plsc.load_gather(offs, [wl])[0]
                end = plsc.load_gather(offs, [wl + 1])[0]
                start = jnp.clip(start, 0, _CAP)
                end = jnp.clip(end, start, _CAP)
                qlo = start // _L

                def grp(q, nl, d=d):
                    pp = (qlo + q) * _L
                    mv = (pp + iota >= start) & (pp + iota < end)
                    cnt = (jnp.minimum(end, pp + _L)
                           - jnp.maximum(start, pp))
                    srcv = jnp.where(mv, order[pl.ds(pp, _L)], 0)
                    lvec = plsc.load_gather(lanes, [srcv], mask=mv)
                    dvec = plsc.load_gather(dests, [srcv], mask=mv)
                    ll = jnp.where(mv, lvec - ws, 0)
                    mi2 = mv.astype(jnp.int32)
                    pos = nl + plsc.cumsum(mi2) - mi2
                    for c in range(d):
                        av = jnp.full((_L,), c // 8, jnp.int32)
                        sv = jnp.full((_L,), c % 8, jnp.int32)
                        vals = plsc.load_gather(wb, [av, sv, ll], mask=mv)
                        plsc.store_scatter(
                            rows, [pos, jnp.full((_L,), c, jnp.int32)],
                            vals, mask=mv)
                    plsc.store_scatter(rdst, [pos], dvec, mask=mv)
                    nl = nl + cnt

                    def do_flush():
                        pltpu.sync_copy(
                            rows,
                            o.at[plsc.Indices(rdst, ignored_value=-1)])
                        return jnp.int32(0)

                    return lax.cond(nl >= _RCAP, do_flush, lambda: nl)

                nwq = (end + _L - 1) // _L - qlo
                return lax.fori_loop(0, nwq, grp, nl)

            def window(k, nl):
                return lax.cond(
                    (k - klo) % 2 == 0,
                    lambda: window_body(0, k, nl),
                    lambda: window_body(1, k, nl))

            nl = lax.fori_loop(klo, khi, window, jnp.int32(0))

            @pl.when(nl > 0)
            def _():
                pltpu.sync_copy(
                    rows, o.at[plsc.Indices(rdst, ignored_value=-1)])

    o = gather3(*tT3, indices_0, indices_1, indices_2)
    return tuple(oo[:, :d] for oo, d in zip(o, dims))


# speedup vs baseline: 1.0143x; 1.0143x over previous
import functools

import jax
import jax.numpy as jnp
from jax import lax
from jax.experimental import pallas as pl
from jax.experimental.pallas import tpu as pltpu
from jax.experimental.pallas import tpu_sc as plsc

_W = 512
_WSH = 9
_CAP = 4096
_RCAP = 80
_L = 16
_NB = 80
_TRASH = _NB - 1


def kernel(table_0, table_1, table_2, indices_0, indices_1, indices_2):
    B = indices_0.shape[0]
    V = table_0.shape[0]
    info = plsc.get_sparse_core_info()
    NC, NS = info.num_cores, info.num_subcores
    NW = NC * NS
    dims = (table_0.shape[1], table_1.shape[1], table_2.shape[1])
    padv = ((V + 127) // 128) * 128
    nwin = (V + _W - 1) // _W
    mesh = plsc.VectorSubcoreMesh(core_axis_name="c", subcore_axis_name="s")

    tT3 = [t.T.reshape(d // 8, 8, V)
           for t, d in zip((table_0, table_1, table_2), dims)]

    scratch = [
        pltpu.VMEM((8, 8, _W), jnp.float32),
        pltpu.VMEM((8, 8, _W), jnp.float32),
        pltpu.VMEM((_CAP,), jnp.int32),
        pltpu.VMEM((_CAP,), jnp.int32),
        pltpu.VMEM((_CAP,), jnp.int32),
        pltpu.VMEM((_NB + _L,), jnp.int32),
        pltpu.VMEM((_NB + _L,), jnp.int32),
        pltpu.VMEM((_NB + _L,), jnp.int32),
        pltpu.VMEM((_L,), jnp.int32),
        pltpu.VMEM((_RCAP + _L, 128), jnp.float32),
        pltpu.VMEM((_RCAP + _L,), jnp.int32),
        pltpu.VMEM((2048,), jnp.int32),
        pltpu.SemaphoreType.DMA,
        pltpu.SemaphoreType.DMA,
    ]

    @functools.partial(
        pl.kernel,
        mesh=mesh,
        out_type=tuple(
            jax.ShapeDtypeStruct((B, 128), jnp.float32) for _ in dims
        ),
        scratch_types=scratch,
        compiler_params=pltpu.CompilerParams(needs_layout_passes=False),
    )
    def gather3(t0, t1, t2, i0, i1, i2, o0, o1, o2,
                win0, win1, lanes, dests, order, hist, offs, curs, tmp,
                rows, rdst, ivb, s0, s1):
        wid = lax.axis_index("s") * NC + lax.axis_index("c")
        iota = lax.iota(jnp.int32, _L)
        wsem = (s0, s1)

        klo = nwin * wid // NW
        khi = nwin * (wid + 1) // NW

        def wstart(k):
            return jnp.minimum(k * _W, padv - _W)

        def runs(w_s):
            tmp[...] = w_s
            prev = plsc.load_gather(tmp, [jnp.maximum(iota - 1, 0)])
            nxt = plsc.load_gather(tmp, [jnp.minimum(iota + 1, _L - 1)])
            first = (iota == 0) | (w_s != prev)
            last = (iota == _L - 1) | (w_s != nxt)
            start = plsc.cummax(jnp.where(first, iota, 0))
            rank = iota - start
            return last, rank

        for n, (t, i, o, d) in enumerate(
            zip((t0, t1, t2), (i0, i1, i2), (o0, o1, o2), dims)
        ):
            nb = d // 8
            llo = wstart(klo)
            lhi = jnp.where(jnp.int32(khi) >= nwin, jnp.int32(V), khi * _W)

            def chunk_a(c8, n_hits, i=i):
                pltpu.sync_copy(i.at[pl.ds(c8 * 2048, 2048)], ivb)

                def grp_a(g, n_hits):
                    ivec = ivb[pl.ds(g * _L, _L)]
                    m = (ivec >= llo) & (ivec < lhi)
                    mi = m.astype(jnp.int32)
                    cs = plsc.cumsum(mi)
                    pos = n_hits + cs - mi
                    m = m & (pos < _CAP)
                    plsc.store_scatter(lanes, [pos], ivec, mask=m)
                    plsc.store_scatter(
                        dests, [pos], iota + (c8 * 2048 + g * _L), mask=m)
                    return n_hits + cs[_L - 1]

                return lax.fori_loop(0, 2048 // _L, grp_a, n_hits)

            n_hits = jnp.minimum(
                lax.fori_loop(0, B // 2048, chunk_a, jnp.int32(0)),
                jnp.int32(_CAP))
            ng = (n_hits + _L - 1) // _L

            def binit(q, _):
                hist[pl.ds(q * _L, _L)] = jnp.zeros((_L,), jnp.int32)
                return _
            lax.fori_loop(0, (_NB + _L) // _L, binit, 0)

            def whist(g, _):
                lvec = lanes[pl.ds(g * _L, _L)]
                valid = iota + g * _L < n_hits
                w = jnp.where(valid, (lvec - llo) >> _WSH, _TRASH)
                w_s, _src = plsc.sort_key_val(w, iota)
                last, rank = runs(w_s)
                plsc.addupdate_scatter(hist, [w_s], rank + 1, mask=last)
                return _
            lax.fori_loop(0, ng, whist, 0)

            def bscan(q, c):
                h = hist[pl.ds(q * _L, _L)]
                cs = plsc.cumsum(h)
                excl = cs - h + c
                offs[pl.ds(q * _L, _L)] = excl
                curs[pl.ds(q * _L, _L)] = excl
                return c + cs[_L - 1]
            lax.fori_loop(0, (_NB + _L) // _L, bscan, jnp.int32(0))

            def wplace(g, _):
                lvec = lanes[pl.ds(g * _L, _L)]
                valid = iota + g * _L < n_hits
                w = jnp.where(valid, (lvec - llo) >> _WSH, _TRASH)
                w_s, src = plsc.sort_key_val(w, iota + g * _L)
                last, rank = runs(w_s)
                base = plsc.load_gather(curs, [w_s])
                plsc.store_scatter(order, [base + rank], src,
                                   mask=w_s != _TRASH)
                plsc.addupdate_scatter(curs, [w_s], rank + 1, mask=last)
                return _
            lax.fori_loop(0, ng, wplace, 0)

            def rinit(q, _):
                rdst[pl.ds(q * _L, _L)] = jnp.full((_L,), -1, jnp.int32)
                return _
            lax.fori_loop(0, (_RCAP + _L) // _L, rinit, 0)

            pltpu.async_copy(
                t.at[:, :, pl.ds(wstart(klo), _W)],
                win0.at[pl.ds(0, nb)], wsem[0])

            def window_body(p, k, nl, t=t, o=o, d=d, nb=nb):
                wb = (win0, win1)[p]
                wn = (win0, win1)[p ^ 1]

                @pl.when(k + 1 < khi)
                def _():
                    pltpu.async_copy(
                        t.at[:, :, pl.ds(wstart(k + 1), _W)],
                        wn.at[pl.ds(0, nb)], wsem[p ^ 1])

                pltpu.make_async_copy(
                    t.at[:, :, pl.ds(0, _W)],
                    wb.at[pl.ds(0, nb)], wsem[p]).wait()
                ws = wstart(k)
                wl = jnp.full((_L,), k - klo, jnp.int32)
                start = plsc.load_gather(offs, [wl])[0]
                end = plsc.load_gather(offs, [wl + 1])[0]
                start = jnp.clip(start, 0, _CAP)
                end = jnp.clip(end, start, _CAP)
                qlo = start // _L

                def grp(q, nl, d=d):
                    pp = (qlo + q) * _L
                    mv = (pp + iota >= start) & (pp + iota < end)
                    cnt = (jnp.minimum(end, pp + _L)
                           - jnp.maximum(start, pp))
                    srcv = jnp.where(mv, order[pl.ds(pp, _L)], 0)
                    lvec = plsc.load_gather(lanes, [srcv], mask=mv)
                    dvec = plsc.load_gather(dests, [srcv], mask=mv)
                    ll = jnp.where(mv, lvec - ws, 0)
                    mi2 = mv.astype(jnp.int32)
                    pos = nl + plsc.cumsum(mi2) - mi2
                    for c in range(d):
                        av = jnp.full((_L,), c // 8, jnp.int32)
                        sv = jnp.full((_L,), c % 8, jnp.int32)
                        vals = plsc.load_gather(wb, [av, sv, ll], mask=mv)
                        plsc.store_scatter(
                            rows, [pos, jnp.full((_L,), c, jnp.int32)],
                            vals, mask=mv)
                    plsc.store_scatter(rdst, [pos], dvec, mask=mv)
                    nl = nl + cnt

                    def do_flush():
                        pltpu.sync_copy(
                            rows,
                            o.at[plsc.Indices(rdst, ignored_value=-1)])
                        return jnp.int32(0)

                    return lax.cond(nl >= _RCAP, do_flush, lambda: nl)

                nwq = (end + _L - 1) // _L - qlo
                return lax.fori_loop(0, nwq, grp, nl)

            def window(k, nl):
                return lax.cond(
                    (k - klo) % 2 == 0,
                    lambda: window_body(0, k, nl),
                    lambda: window_body(1, k, nl))

            nl = lax.fori_loop(klo, khi, window, jnp.int32(0))

            @pl.when(nl > 0)
            def _():
                pltpu.sync_copy(
                    rows, o.at[plsc.Indices(rdst, ignored_value=-1)])

    o = gather3(*tT3, indices_0, indices_1, indices_2)
    return tuple(oo[:, :d] for oo, d in zip(o, dims))


# speedup vs baseline: 1.0290x; 1.0145x over previous
import functools

import jax
import jax.numpy as jnp
from jax import lax
from jax.experimental import pallas as pl
from jax.experimental.pallas import tpu as pltpu
from jax.experimental.pallas import tpu_sc as plsc

_W = 512
_WSH = 9
_CAP = 4096
_RCAP = 80
_L = 16
_NB = 80
_TRASH = _NB - 1


def kernel(table_0, table_1, table_2, indices_0, indices_1, indices_2):
    B = indices_0.shape[0]
    V = table_0.shape[0]
    info = plsc.get_sparse_core_info()
    NC, NS = info.num_cores, info.num_subcores
    NW = NC * NS
    dims = (table_0.shape[1], table_1.shape[1], table_2.shape[1])
    padv = ((V + 127) // 128) * 128
    nwin = (V + _W - 1) // _W
    mesh = plsc.VectorSubcoreMesh(core_axis_name="c", subcore_axis_name="s")

    tT3 = [t.T.reshape(d // 8, 8, V)
           for t, d in zip((table_0, table_1, table_2), dims)]

    scratch = [
        pltpu.VMEM((8, 8, _W), jnp.float32),
        pltpu.VMEM((8, 8, _W), jnp.float32),
        pltpu.VMEM((_CAP,), jnp.int32),
        pltpu.VMEM((_CAP,), jnp.int32),
        pltpu.VMEM((_CAP,), jnp.int32),
        pltpu.VMEM((_NB + _L,), jnp.int32),
        pltpu.VMEM((_NB + _L,), jnp.int32),
        pltpu.VMEM((_NB + _L,), jnp.int32),
        pltpu.VMEM((_L,), jnp.int32),
        pltpu.VMEM((_RCAP + _L, 128), jnp.float32),
        pltpu.VMEM((_RCAP + _L,), jnp.int32),
        pltpu.VMEM((2048,), jnp.int32),
        pltpu.SemaphoreType.DMA,
        pltpu.SemaphoreType.DMA,
    ]

    @functools.partial(
        pl.kernel,
        mesh=mesh,
        out_type=tuple(
            jax.ShapeDtypeStruct((B, 128), jnp.float32) for _ in dims
        ),
        scratch_types=scratch,
        compiler_params=pltpu.CompilerParams(needs_layout_passes=False),
    )
    def gather3(t0, t1, t2, i0, i1, i2, o0, o1, o2,
                win0, win1, lanes, dests, order, hist, offs, curs, tmp,
                rows, rdst, ivb, s0, s1):
        wid = lax.axis_index("s") * NC + lax.axis_index("c")
        iota = lax.iota(jnp.int32, _L)
        wsem = (s0, s1)

        klo = nwin * wid // NW
        khi = nwin * (wid + 1) // NW

        def wstart(k):
            return jnp.minimum(k * _W, padv - _W)

        def runs(w_s):
            tmp[...] = w_s
            prev = plsc.load_gather(tmp, [jnp.maximum(iota - 1, 0)])
            nxt = plsc.load_gather(tmp, [jnp.minimum(iota + 1, _L - 1)])
            first = (iota == 0) | (w_s != prev)
            last = (iota == _L - 1) | (w_s != nxt)
            start = plsc.cummax(jnp.where(first, iota, 0))
            rank = iota - start
            return last, rank

        for n, (t, i, o, d) in enumerate(
            zip((t0, t1, t2), (i0, i1, i2), (o0, o1, o2), dims)
        ):
            nb = d // 8
            llo = wstart(klo)
            lhi = jnp.where(jnp.int32(khi) >= nwin, jnp.int32(V), khi * _W)

            def chunk_a(c8, n_hits, i=i):
                pltpu.sync_copy(i.at[pl.ds(c8 * 2048, 2048)], ivb)

                def grp_a(g, n_hits):
                    ivec = ivb[pl.ds(g * _L, _L)]
                    m = (ivec >= llo) & (ivec < lhi)
                    mi = m.astype(jnp.int32)
                    pos = n_hits + plsc.cumsum(mi) - mi
                    m = m & (pos < _CAP)
                    plsc.store_scatter(lanes, [pos], ivec, mask=m)
                    plsc.store_scatter(
                        dests, [pos], iota + (c8 * 2048 + g * _L), mask=m)
                    return n_hits + plsc.all_reduce_population_count(m)[0]

                return lax.fori_loop(0, 2048 // _L, grp_a, n_hits)

            n_hits = lax.fori_loop(0, B // 2048, chunk_a, jnp.int32(0))
            ng = (n_hits + _L - 1) // _L

            def binit(q, _):
                hist[pl.ds(q * _L, _L)] = jnp.zeros((_L,), jnp.int32)
                return _
            lax.fori_loop(0, (_NB + _L) // _L, binit, 0)

            def whist(g, _):
                lvec = lanes[pl.ds(g * _L, _L)]
                valid = iota + g * _L < n_hits
                w = jnp.where(valid, (lvec - llo) >> _WSH, _TRASH)
                w_s, _src = plsc.sort_key_val(w, iota)
                last, rank = runs(w_s)
                plsc.addupdate_scatter(hist, [w_s], rank + 1, mask=last)
                return _
            lax.fori_loop(0, ng, whist, 0)

            def bscan(q, c):
                h = hist[pl.ds(q * _L, _L)]
                cs = plsc.cumsum(h)
                excl = cs - h + c
                offs[pl.ds(q * _L, _L)] = excl
                curs[pl.ds(q * _L, _L)] = excl
                return c + cs[_L - 1]
            lax.fori_loop(0, (_NB + _L) // _L, bscan, jnp.int32(0))

            def wplace(g, _):
                lvec = lanes[pl.ds(g * _L, _L)]
                valid = iota + g * _L < n_hits
                w = jnp.where(valid, (lvec - llo) >> _WSH, _TRASH)
                w_s, src = plsc.sort_key_val(w, iota + g * _L)
                last, rank = runs(w_s)
                base = plsc.load_gather(curs, [w_s])
                plsc.store_scatter(order, [base + rank], src,
                                   mask=w_s != _TRASH)
                plsc.addupdate_scatter(curs, [w_s], rank + 1, mask=last)
                return _
            lax.fori_loop(0, ng, wplace, 0)

            def rinit(q, _):
                rdst[pl.ds(q * _L, _L)] = jnp.full((_L,), -1, jnp.int32)
                return _
            lax.fori_loop(0, (_RCAP + _L) // _L, rinit, 0)

            pltpu.async_copy(
                t.at[:, :, pl.ds(wstart(klo), _W)],
                win0.at[pl.ds(0, nb)], wsem[0])

            def window_body(p, k, nl, t=t, o=o, d=d, nb=nb):
                wb = (win0, win1)[p]
                wn = (win0, win1)[p ^ 1]

                @pl.when(k + 1 < khi)
                def _():
                    pltpu.async_copy(
                        t.at[:, :, pl.ds(wstart(k + 1), _W)],
                        wn.at[pl.ds(0, nb)], wsem[p ^ 1])

                pltpu.make_async_copy(
                    t.at[:, :, pl.ds(0, _W)],
                    wb.at[pl.ds(0, nb)], wsem[p]).wait()
                ws = wstart(k)
                wl = jnp.full((_L,), k - klo, jnp.int32)
                start = plsc.load_gather(offs, [wl])[0]
                end = plsc.load_gather(offs, [wl + 1])[0]
                start = jnp.clip(start, 0, _CAP)
                end = jnp.clip(end, start, _CAP)
                qlo = start // _L

                def grp(q, nl, d=d):
                    pp = (qlo + q) * _L
                    mv = (pp + iota >= start) & (pp + iota < end)
                    cnt = (jnp.minimum(end, pp + _L)
                           - jnp.maximum(start, pp))
                    srcv = jnp.where(mv, order[pl.ds(pp, _L)], 0)
                    lvec = plsc.load_gather(lanes, [srcv], mask=mv)
                    dvec = plsc.load_gather(dests, [srcv], mask=mv)
                    ll = jnp.where(mv, lvec - ws, 0)
                    mi2 = mv.astype(jnp.int32)
                    pos = nl + plsc.cumsum(mi2) - mi2
                    for c in range(d):
                        av = jnp.full((_L,), c // 8, jnp.int32)
                        sv = jnp.full((_L,), c % 8, jnp.int32)
                        vals = plsc.load_gather(wb, [av, sv, ll], mask=mv)
                        plsc.store_scatter(
                            rows, [pos, jnp.full((_L,), c, jnp.int32)],
                            vals, mask=mv)
                    plsc.store_scatter(rdst, [pos], dvec, mask=mv)
                    nl = nl + cnt

                    def do_flush():
                        pltpu.sync_copy(
                            rows,
                            o.at[plsc.Indices(rdst, ignored_value=-1)])
                        return jnp.int32(0)

                    return lax.cond(nl >= _RCAP, do_flush, lambda: nl)

                nwq = (end + _L - 1) // _L - qlo
                return lax.fori_loop(0, nwq, grp, nl)

            def window(k, nl):
                return lax.cond(
                    (k - klo) % 2 == 0,
                    lambda: window_body(0, k, nl),
                    lambda: window_body(1, k, nl))

            nl = lax.fori_loop(klo, khi, window, jnp.int32(0))

            @pl.when(nl > 0)
            def _():
                pltpu.sync_copy(
                    rows, o.at[plsc.Indices(rdst, ignored_value=-1)])

    o = gather3(*tT3, indices_0, indices_1, indices_2)
    return tuple(oo[:, :d] for oo, d in zip(o, dims))
